# baseline (speedup 1.0000x reference)
import functools

import jax
import jax.numpy as jnp
from jax import lax
from jax.experimental import pallas as pl
from jax.experimental.pallas import tpu as pltpu
from jax.experimental.pallas import tpu_sc as plsc

_N = 10000
_E = 320000
_D = 128
_H = _D // 2
_NC = 2
_NS = 16
_NW = _NC * _NS
_CHUNK = 128
_NPAD = 10240
_RPT = _NPAD // _NS

_EPT2 = _E // _NS
_CK2 = 88
_CPT2 = 232
_PAD2 = _CPT2 * _CK2 - _EPT2
_WCH = 80

_EPT = _E // _NW
_CPT = -(-_EPT // _CHUNK)
_PAD1 = _CPT * _CHUNK - _EPT


def _mesh():
    return plsc.VectorSubcoreMesh(core_axis_name="c", subcore_axis_name="s")


@functools.partial(
    pl.kernel,
    mesh=_mesh(),
    out_type=jax.ShapeDtypeStruct((_NC, _NPAD, _H), jnp.float32),
    compiler_params=pltpu.CompilerParams(use_tc_tiling_on_sc=False),
    scratch_types=[
        pltpu.VMEM((_CPT2, _CK2), jnp.int32),
        pltpu.VMEM((_CPT2, _CK2), jnp.int32),
        pltpu.VMEM((8, _CK2, _H), jnp.float32),
        pltpu.VMEM_SHARED((_NPAD, _H), jnp.float32),
        pltpu.SemaphoreType.DMA((8,)),
    ],
)
def _round_k(xs_hbm, gidx_hbm, sidx_hbm, z_hbm, out_hbm,
             gidx_v, sidx_v, rowsbuf, acc, sems):
    c = lax.axis_index("c")
    s = lax.axis_index("s")
    pltpu.sync_copy(gidx_hbm.at[c, s], gidx_v)
    pltpu.sync_copy(sidx_hbm.at[s], sidx_v)
    zb = s * _RPT
    for k in range(_RPT // _WCH):
        r = zb + k * _WCH
        pltpu.sync_copy(z_hbm.at[pl.ds(r, _WCH)], rowsbuf.at[0, pl.ds(0, _WCH)])
        pltpu.sync_copy(rowsbuf.at[0, pl.ds(0, _WCH)], acc.at[pl.ds(r, _WCH)])
    plsc.subcore_barrier()

    def gather(i, b):
        pltpu.async_copy(xs_hbm.at[gidx_v.at[i]], rowsbuf.at[b], sems.at[b])

    def wait_gather(i, b):
        pltpu.make_async_copy(
            xs_hbm.at[gidx_v.at[i]], rowsbuf.at[b], sems.at[b]).wait()

    def scatter(i, b):
        pltpu.async_copy(
            rowsbuf.at[b], acc.at[sidx_v.at[i]], sems.at[b], add=True)

    def wait_scatter(i, b):
        pltpu.make_async_copy(
            rowsbuf.at[b], acc.at[sidx_v.at[i]], sems.at[b]).wait()

    for j in range(4):
        gather(j, j)
    for j in range(4):
        wait_gather(j, j)
        scatter(j, j)
        gather(j + 4, j + 4)

    def body(ii, carry):
        for j in range(8):
            i = ii * 8 + j + 4
            b = (j + 4) % 8
            wait_gather(i, b)
            scatter(i, b)
            wait_scatter(i - 4, j)
            gather(i + 4, j)
        return carry

    lax.fori_loop(0, (_CPT2 - 8) // 8, body, 0)
    for t in range(4):
        i = _CPT2 - 4 + t
        b = (t + 4) % 8
        wait_gather(i, b)
        scatter(i, b)
        wait_scatter(i - 4, t)
    for t in range(4):
        wait_scatter(_CPT2 - 4 + t, (t + 4) % 8)
    plsc.subcore_barrier()
    for k in range(_RPT // _WCH):
        r = zb + k * _WCH
        pltpu.sync_copy(acc.at[pl.ds(r, _WCH)], rowsbuf.at[0, pl.ds(0, _WCH)])
        pltpu.sync_copy(rowsbuf.at[0, pl.ds(0, _WCH)],
                        out_hbm.at[c, pl.ds(r, _WCH)])


@functools.partial(
    pl.kernel,
    mesh=_mesh(),
    out_type=jax.ShapeDtypeStruct((_NC, _NPAD, 16), jnp.float32),
    compiler_params=pltpu.CompilerParams(use_tc_tiling_on_sc=False),
    scratch_types=[
        pltpu.VMEM((_CPT, _CHUNK), jnp.int32),
        pltpu.VMEM((_CHUNK, 16), jnp.float32),
        pltpu.VMEM_SHARED((_NPAD, 16), jnp.float32),
    ],
)
def _count_k(sidx_hbm, z16_hbm, ones_hbm, out_hbm, sv, ones_v, acc):
    c = lax.axis_index("c")
    s = lax.axis_index("s")
    pltpu.sync_copy(sidx_hbm.at[c, s], sv)
    zb = s * _RPT
    for k in range(_RPT // _CHUNK):
        r = zb + k * _CHUNK
        pltpu.sync_copy(z16_hbm.at[pl.ds(r, _CHUNK)], ones_v)
        pltpu.sync_copy(ones_v, acc.at[pl.ds(r, _CHUNK)])
    pltpu.sync_copy(ones_hbm, ones_v)
    plsc.subcore_barrier()

    def body(i, carry):
        pltpu.sync_copy(ones_v, acc.at[sv.at[i]], add=True)
        return carry

    lax.fori_loop(0, _CPT, body, 0)
    plsc.subcore_barrier()
    for k in range(_RPT // _CHUNK):
        r = zb + k * _CHUNK
        pltpu.sync_copy(acc.at[pl.ds(r, _CHUNK)], ones_v)
        pltpu.sync_copy(ones_v, out_hbm.at[c, pl.ds(r, _CHUNK)])


_CB = 2000


def _combine_body(p_ref, c_ref, o_ref):
    cs = c_ref[0, :, 0:1] + c_ref[1, :, 0:1]
    inv = 1.0 / jnp.maximum(cs, 1.0)
    o_ref[...] = jnp.concatenate([p_ref[0] * inv, p_ref[1] * inv], axis=1)


def _combine(p, cnt):
    return pl.pallas_call(
        _combine_body,
        grid=(_N // _CB,),
        in_specs=[
            pl.BlockSpec((_NC, _CB, _H), lambda i: (0, i, 0)),
            pl.BlockSpec((_NC, _CB, 16), lambda i: (0, i, 0)),
        ],
        out_specs=pl.BlockSpec((_CB, _D), lambda i: (i, 0)),
        out_shape=jax.ShapeDtypeStruct((_N, _D), jnp.float32),
    )(p, cnt)


def _pack(idx, nway, ept, padvals, chunk=_CHUNK):
    body = idx.reshape(nway, ept)
    cpt = (ept + padvals.shape[1]) // chunk
    return jnp.concatenate([body, padvals], axis=1).reshape(nway, cpt, chunk)


def kernel(topic_one_hot, edge_index):
    x = topic_one_hot
    src = edge_index[0]
    dst = edge_index[1]

    lin1 = (jnp.arange(_NW, dtype=jnp.int32)[:, None] * _PAD1
            + jnp.arange(_PAD1, dtype=jnp.int32)[None, :])
    spad1 = _N + lin1 % (_NPAD - _N)
    sf32 = _pack(dst, _NW, _EPT, spad1).reshape(_NC, _NS, _CPT, _CHUNK)
    sr32 = _pack(src, _NW, _EPT, spad1).reshape(_NC, _NS, _CPT, _CHUNK)

    lin2 = (jnp.arange(_NS, dtype=jnp.int32)[:, None] * _PAD2
            + jnp.arange(_PAD2, dtype=jnp.int32)[None, :])
    gpad2 = (lin2 * 97) % _N
    spad2 = _N + lin2 % (_NPAD - _N)
    gf = _pack(src, _NS, _EPT2, gpad2, _CK2)
    sf = _pack(dst, _NS, _EPT2, spad2, _CK2)
    gr = _pack(dst, _NS, _EPT2, gpad2, _CK2)
    sr = _pack(src, _NS, _EPT2, spad2, _CK2)
    gf2 = jnp.stack([2 * gf, 2 * gf + 1])
    gr2 = jnp.stack([2 * gr, 2 * gr + 1])

    z = jnp.zeros((_NPAD, _H), jnp.float32)
    z16 = jnp.zeros((_NPAD, 16), jnp.float32)
    ones = jnp.ones((_CHUNK, 16), jnp.float32)

    cntf = _count_k(sf32, z16, ones)
    cntr = _count_k(sr32, z16, ones)

    p = _round_k(x.reshape(2 * _N, _H), gf2, sf, z)
    h1 = _combine(p, cntf)
    p = _round_k(x.reshape(2 * _N, _H), gr2, sr, z)
    r1 = _combine(p, cntr)
    p = _round_k(h1.reshape(2 * _N, _H), gf2, sf, z)
    h2 = _combine(p, cntf)
    p = _round_k(r1.reshape(2 * _N, _H), gr2, sr, z)
    r2 = _combine(p, cntr)
    return (h1, h2, r1, r2)


# speedup vs baseline: 1.1334x; 1.1334x over previous
import functools

import jax
import jax.numpy as jnp
from jax import lax
from jax.experimental import pallas as pl
from jax.experimental.pallas import tpu as pltpu
from jax.experimental.pallas import tpu_sc as plsc

_N = 10000
_E = 320000
_D = 128
_H = _D // 2
_NC = 2
_NS = 16
_NW = _NC * _NS
_CHUNK = 128
_NPAD = 10240
_RPT = _NPAD // _NS

_EPT2 = _E // _NS
_CK2 = 128
_CPT2 = 160
_PAD2 = _CPT2 * _CK2 - _EPT2
_WCH = 128

_EPT = _E // _NW
_CPT = -(-_EPT // _CHUNK)
_PAD1 = _CPT * _CHUNK - _EPT


def _mesh():
    return plsc.VectorSubcoreMesh(core_axis_name="c", subcore_axis_name="s")


@functools.partial(
    pl.kernel,
    mesh=_mesh(),
    out_type=jax.ShapeDtypeStruct((_NC, _NPAD, _H), jnp.float32),
    compiler_params=pltpu.CompilerParams(use_tc_tiling_on_sc=False),
    scratch_types=[
        pltpu.VMEM((_CPT2, _CK2), jnp.int32),
        pltpu.VMEM((_CPT2, _CK2), jnp.int32),
        pltpu.VMEM((4, _CK2, _H), jnp.float32),
        pltpu.VMEM_SHARED((_NPAD, _H), jnp.float32),
        pltpu.SemaphoreType.DMA((4,)),
    ],
)
def _round_k(xs_hbm, gidx_hbm, sidx_hbm, z_hbm, out_hbm,
             gidx_v, sidx_v, rowsbuf, acc, sems):
    c = lax.axis_index("c")
    s = lax.axis_index("s")
    pltpu.sync_copy(gidx_hbm.at[c, s], gidx_v)
    pltpu.sync_copy(sidx_hbm.at[s], sidx_v)
    zb = s * _RPT
    for k in range(_RPT // _WCH):
        r = zb + k * _WCH
        pltpu.sync_copy(z_hbm.at[pl.ds(r, _WCH)], rowsbuf.at[0, pl.ds(0, _WCH)])
        pltpu.sync_copy(rowsbuf.at[0, pl.ds(0, _WCH)], acc.at[pl.ds(r, _WCH)])
    plsc.subcore_barrier()

    def gather(i, b):
        pltpu.async_copy(xs_hbm.at[gidx_v.at[i]], rowsbuf.at[b], sems.at[b])

    def wait_gather(i, b):
        pltpu.make_async_copy(
            xs_hbm.at[gidx_v.at[i]], rowsbuf.at[b], sems.at[b]).wait()

    def scatter(i, b):
        pltpu.async_copy(
            rowsbuf.at[b], acc.at[sidx_v.at[i]], sems.at[b], add=True)

    def wait_scatter(i, b):
        pltpu.make_async_copy(
            rowsbuf.at[b], acc.at[sidx_v.at[i]], sems.at[b]).wait()

    for j in range(4):
        gather(j, j)

    def body(ii, carry):
        for j in range(4):
            i = ii * 4 + j
            wait_gather(i, j)
            scatter(i, j)
            wait_scatter(i, j)
            gather(i + 4, j)
        return carry

    lax.fori_loop(0, _CPT2 // 4 - 1, body, 0)
    for j in range(4):
        i = _CPT2 - 4 + j
        wait_gather(i, j)
        pltpu.sync_copy(rowsbuf.at[j], acc.at[sidx_v.at[i]], add=True)
    plsc.subcore_barrier()
    for k in range(_RPT // _WCH):
        r = zb + k * _WCH
        pltpu.sync_copy(acc.at[pl.ds(r, _WCH)], rowsbuf.at[0, pl.ds(0, _WCH)])
        pltpu.sync_copy(rowsbuf.at[0, pl.ds(0, _WCH)],
                        out_hbm.at[c, pl.ds(r, _WCH)])


@functools.partial(
    pl.kernel,
    mesh=_mesh(),
    out_type=jax.ShapeDtypeStruct((_NC, _NPAD, 16), jnp.float32),
    compiler_params=pltpu.CompilerParams(use_tc_tiling_on_sc=False),
    scratch_types=[
        pltpu.VMEM((_CPT, _CHUNK), jnp.int32),
        pltpu.VMEM((_CHUNK, 16), jnp.float32),
        pltpu.VMEM_SHARED((_NPAD, 16), jnp.float32),
    ],
)
def _count_k(sidx_hbm, z16_hbm, ones_hbm, out_hbm, sv, ones_v, acc):
    c = lax.axis_index("c")
    s = lax.axis_index("s")
    pltpu.sync_copy(sidx_hbm.at[c, s], sv)
    zb = s * _RPT
    for k in range(_RPT // _CHUNK):
        r = zb + k * _CHUNK
        pltpu.sync_copy(z16_hbm.at[pl.ds(r, _CHUNK)], ones_v)
        pltpu.sync_copy(ones_v, acc.at[pl.ds(r, _CHUNK)])
    pltpu.sync_copy(ones_hbm, ones_v)
    plsc.subcore_barrier()

    def body(i, carry):
        pltpu.sync_copy(ones_v, acc.at[sv.at[i]], add=True)
        return carry

    lax.fori_loop(0, _CPT, body, 0)
    plsc.subcore_barrier()
    for k in range(_RPT // _CHUNK):
        r = zb + k * _CHUNK
        pltpu.sync_copy(acc.at[pl.ds(r, _CHUNK)], ones_v)
        pltpu.sync_copy(ones_v, out_hbm.at[c, pl.ds(r, _CHUNK)])


_CB = 2000


def _combine_body(p_ref, c_ref, o_ref):
    cs = c_ref[0, :, 0:1] + c_ref[1, :, 0:1]
    inv = 1.0 / jnp.maximum(cs, 1.0)
    o_ref[...] = jnp.concatenate([p_ref[0] * inv, p_ref[1] * inv], axis=1)


def _combine(p, cnt):
    return pl.pallas_call(
        _combine_body,
        grid=(_N // _CB,),
        in_specs=[
            pl.BlockSpec((_NC, _CB, _H), lambda i: (0, i, 0)),
            pl.BlockSpec((_NC, _CB, 16), lambda i: (0, i, 0)),
        ],
        out_specs=pl.BlockSpec((_CB, _D), lambda i: (i, 0)),
        out_shape=jax.ShapeDtypeStruct((_N, _D), jnp.float32),
    )(p, cnt)


def _pack(idx, nway, ept, padvals, chunk=_CHUNK):
    body = idx.reshape(nway, ept)
    cpt = (ept + padvals.shape[1]) // chunk
    return jnp.concatenate([body, padvals], axis=1).reshape(nway, cpt, chunk)


def kernel(topic_one_hot, edge_index):
    x = topic_one_hot
    src = edge_index[0]
    dst = edge_index[1]

    lin1 = (jnp.arange(_NW, dtype=jnp.int32)[:, None] * _PAD1
            + jnp.arange(_PAD1, dtype=jnp.int32)[None, :])
    spad1 = _N + lin1 % (_NPAD - _N)
    sf32 = _pack(dst, _NW, _EPT, spad1).reshape(_NC, _NS, _CPT, _CHUNK)
    sr32 = _pack(src, _NW, _EPT, spad1).reshape(_NC, _NS, _CPT, _CHUNK)

    lin2 = (jnp.arange(_NS, dtype=jnp.int32)[:, None] * _PAD2
            + jnp.arange(_PAD2, dtype=jnp.int32)[None, :])
    gpad2 = (lin2 * 97) % _N
    spad2 = _N + lin2 % (_NPAD - _N)
    gf = _pack(src, _NS, _EPT2, gpad2, _CK2)
    sf = _pack(dst, _NS, _EPT2, spad2, _CK2)
    gr = _pack(dst, _NS, _EPT2, gpad2, _CK2)
    sr = _pack(src, _NS, _EPT2, spad2, _CK2)
    gf2 = jnp.stack([2 * gf, 2 * gf + 1])
    gr2 = jnp.stack([2 * gr, 2 * gr + 1])

    z = jnp.zeros((_NPAD, _H), jnp.float32)
    z16 = jnp.zeros((_NPAD, 16), jnp.float32)
    ones = jnp.ones((_CHUNK, 16), jnp.float32)

    cntf = _count_k(sf32, z16, ones)
    cntr = _count_k(sr32, z16, ones)

    p = _round_k(x.reshape(2 * _N, _H), gf2, sf, z)
    h1 = _combine(p, cntf)
    p = _round_k(x.reshape(2 * _N, _H), gr2, sr, z)
    r1 = _combine(p, cntr)
    p = _round_k(h1.reshape(2 * _N, _H), gf2, sf, z)
    h2 = _combine(p, cntf)
    p = _round_k(r1.reshape(2 * _N, _H), gr2, sr, z)
    r2 = _combine(p, cntr)
    return (h1, h2, r1, r2)
